# baseline (speedup 1.0000x reference)
import jax
import jax.numpy as jnp
from jax.experimental import pallas as pl
from jax.experimental.pallas import tpu as pltpu

_NEIGHBORS = (0, 5, 11, 17)
_TIME_LEN = 9
_E = 16
_D = 128
_NB = 4
_FLAT = _TIME_LEN * _NB * _D
_H = 512
_KL_W = 0.01
_NCORES = 2
_EPC = _E // _NCORES


def _body(t_ref, x_ref, wg_ref, bg_ref, w1_ref, b1_ref, w2_ref, b2_ref,
          w3_ref, b3_ref, wt_ref, part_ref, idx_ref,
          w1buf, w2buf, w3buf, flat_scr, flatb_scr, wgv, wtv, bgv,
          b1v, b2v, b3v, idx_scr, part_scr, t_smem,
          sem_w1, sem_w2, sem_w3, sem_misc):
    core = jax.lax.axis_index("core")
    dt_half = _TIME_LEN // 2
    e0 = core * _EPC

    pending = {}

    def start_weights(e_local, slot):
        e = e0 + e_local
        cs = (pltpu.make_async_copy(w1_ref.at[e], w1buf.at[slot],
                                    sem_w1.at[slot]),
              pltpu.make_async_copy(w2_ref.at[e], w2buf.at[slot],
                                    sem_w2.at[slot]),
              pltpu.make_async_copy(w3_ref.at[e], w3buf.at[slot],
                                    sem_w3.at[slot]))
        for c in cs:
            c.start()
        pending[slot] = cs

    def wait_weights(slot):
        for c in pending[slot]:
            c.wait()

    start_weights(0, 0)

    cp_t = pltpu.make_async_copy(t_ref, t_smem, sem_misc)
    cp_t.start()
    smalls = [pltpu.make_async_copy(src, dst, sem_misc)
              for src, dst in ((wg_ref, wgv), (wt_ref, wtv), (bg_ref, bgv),
                               (b1_ref, b1v), (b2_ref, b2v), (b3_ref, b3v))]
    for c in smalls:
        c.start()
    cp_t.wait()
    t0 = t_smem[0] - dt_half

    gathers = []
    for ti in range(_TIME_LEN):
        for nb in range(_NB):
            k = ti * _NB + nb
            c = pltpu.make_async_copy(
                x_ref.at[:, t0 + ti, _NEIGHBORS[nb], :],
                flat_scr.at[:, pl.ds(k * _D, _D)], sem_misc)
            c.start()
            gathers.append(c)
    for c in gathers:
        c.wait()
    for c in smalls:
        c.wait()

    flat = flat_scr[...]
    flatb = flat.astype(jnp.bfloat16)
    flatb_scr[...] = flatb
    B = flat.shape[0]

    logits = jnp.dot(flatb, wgv[...].astype(jnp.bfloat16),
                     preferred_element_type=jnp.float32) + bgv[...]
    m = jnp.max(logits, axis=-1, keepdims=True)
    ex = jnp.exp(logits - m)
    g = ex / jnp.sum(ex, axis=-1, keepdims=True)

    center_b = flatb[:, dt_half * _NB * _D:(dt_half + 1) * _NB * _D]
    tgt = jnp.dot(center_b, wtv[...].astype(jnp.bfloat16),
                  preferred_element_type=jnp.float32)

    @pl.when(core == 0)
    def _write_idx():
        lane = jax.lax.broadcasted_iota(jnp.int32, logits.shape, 1)
        is_max = logits == jnp.max(logits, axis=-1, keepdims=True)
        idx_scr[0, :] = jnp.min(jnp.where(is_max, lane, _E), axis=-1)
        pltpu.make_async_copy(idx_scr, idx_ref, sem_misc).start()

    acc = jnp.zeros((B,), jnp.float32)
    lane16 = jax.lax.broadcasted_iota(jnp.int32, (1, _E), 1)
    for e_local in range(_EPC):
        slot = e_local % 2
        wait_weights(slot)
        if e_local + 1 < _EPC:
            start_weights(e_local + 1, 1 - slot)
        h = jnp.dot(flatb, w1buf[slot].astype(jnp.bfloat16),
                    preferred_element_type=jnp.float32) \
            + b1v[pl.ds(e0 + e_local, 1), :]
        h = jnp.maximum(h, 0.0)
        h = jnp.dot(h.astype(jnp.bfloat16), w2buf[slot].astype(jnp.bfloat16),
                    preferred_element_type=jnp.float32) \
            + b2v[pl.ds(e0 + e_local, 1), :]
        h = jnp.maximum(h, 0.0)
        y = jnp.dot(h.astype(jnp.bfloat16), w3buf[slot].astype(jnp.bfloat16),
                    preferred_element_type=jnp.float32) \
            + b3v[pl.ds(e0 + e_local, 1), :]
        mse_e = jnp.mean((y - tgt) ** 2, axis=-1)
        g_col = jnp.sum(
            jnp.where(lane16 == e0 + e_local, g, 0.0), axis=-1)
        acc = acc + g_col * mse_e

    s = jnp.sum(acc)

    @pl.when(core == 0)
    def _add_kl():
        usage = jnp.sum(g, axis=0, keepdims=True) / B
        kl = jnp.sum(usage * (jnp.log(usage + 1e-9) - jnp.log(1.0 / _E)))
        part_scr[...] = jnp.reshape(s + B * _KL_W * kl, (1, 1))

    @pl.when(core != 0)
    def _no_kl():
        part_scr[...] = jnp.reshape(s, (1, 1))

    cp_out = pltpu.make_async_copy(part_scr, part_ref.at[pl.ds(core, 1), :],
                                   sem_misc)
    cp_out.start()
    cp_out.wait()

    @pl.when(core == 0)
    def _wait_idx():
        pltpu.make_async_copy(idx_scr, idx_ref, sem_misc).wait()


def kernel(x, t, Wg, bg, W1, b1, W2, b2, W3, b3, Wt):
    B = x.shape[0]
    t_arr = jnp.asarray(t, jnp.int32).reshape(1)
    bg2 = bg.reshape(1, _E)

    mesh = pltpu.create_tensorcore_mesh("core", num_cores=_NCORES)
    run = pl.kernel(
        _body,
        out_type=[
            jax.ShapeDtypeStruct((_NCORES, 1), jnp.float32),
            jax.ShapeDtypeStruct((1, B), jnp.int32),
        ],
        mesh=mesh,
        scratch_types=[
            pltpu.VMEM((2, _FLAT, _H), jnp.float32),
            pltpu.VMEM((2, _H, _H), jnp.float32),
            pltpu.VMEM((2, _H, _D), jnp.float32),
            pltpu.VMEM((B, _FLAT), jnp.float32),
            pltpu.VMEM((B, _FLAT), jnp.bfloat16),
            pltpu.VMEM((_FLAT, _E), jnp.float32),
            pltpu.VMEM((_NB * _D, _D), jnp.float32),
            pltpu.VMEM((1, _E), jnp.float32),
            pltpu.VMEM((_E, _H), jnp.float32),
            pltpu.VMEM((_E, _H), jnp.float32),
            pltpu.VMEM((_E, _D), jnp.float32),
            pltpu.VMEM((1, B), jnp.int32),
            pltpu.VMEM((1, 1), jnp.float32),
            pltpu.SMEM((1,), jnp.int32),
            pltpu.SemaphoreType.DMA((2,)),
            pltpu.SemaphoreType.DMA((2,)),
            pltpu.SemaphoreType.DMA((2,)),
            pltpu.SemaphoreType.DMA,
        ],
    )
    part, idx = run(t_arr, x, Wg, bg2, W1, b1, W2, b2, W3, b3, Wt)
    loss = (part[0, 0] + part[1, 0]) / B
    return loss, idx.reshape(B)


# speedup vs baseline: 1.2192x; 1.2192x over previous
import jax
import jax.numpy as jnp
from jax.experimental import pallas as pl
from jax.experimental.pallas import tpu as pltpu

_NEIGHBORS = (0, 5, 11, 17)
_TIME_LEN = 9
_E = 16
_D = 128
_NB = 4
_FLAT = _TIME_LEN * _NB * _D
_H = 512
_HH = _H // 2
_KL_W = 0.01


def _moe_kernel(t_ref, x_ref, wg_ref, bg_ref, w1_ref, b1_ref, w2_ref, b2_ref,
                w3_ref, b3_ref, wt_ref, loss_ref, idx_ref,
                flat_scr, flatb_scr, h_scr, g_scr, mse_scr, tgt_scr, sem):
    e = pl.program_id(0)
    j = pl.program_id(1)
    dt_half = _TIME_LEN // 2

    @pl.when((e == 0) & (j == 0))
    def _gather_and_route():
        t0 = t_ref[0] - dt_half
        copies = []
        for ti in range(_TIME_LEN):
            for nb in range(_NB):
                jj = _NEIGHBORS[nb]
                k = ti * _NB + nb
                c = pltpu.make_async_copy(
                    x_ref.at[:, t0 + ti, jj, :],
                    flat_scr.at[:, pl.ds(k * _D, _D)],
                    sem,
                )
                c.start()
                copies.append(c)
        for c in copies:
            c.wait()

        flat = flat_scr[...]
        flatb = flat.astype(jnp.bfloat16)
        flatb_scr[...] = flatb

        logits = jnp.dot(flatb, wg_ref[...].astype(jnp.bfloat16),
                         preferred_element_type=jnp.float32) + bg_ref[...]
        m = jnp.max(logits, axis=-1, keepdims=True)
        ex = jnp.exp(logits - m)
        g = ex / jnp.sum(ex, axis=-1, keepdims=True)
        g_scr[...] = g

        lane = jax.lax.broadcasted_iota(jnp.int32, logits.shape, 1)
        is_max = logits == jnp.max(logits, axis=-1, keepdims=True)
        idx = jnp.min(jnp.where(is_max, lane, _E), axis=-1)
        idx_ref[0, :] = idx

        center = flat_scr[:, pl.ds(dt_half * _NB * _D, _NB * _D)]
        tgt_scr[...] = jnp.dot(center.astype(jnp.bfloat16),
                               wt_ref[...].astype(jnp.bfloat16),
                               preferred_element_type=jnp.float32)
        mse_scr[...] = jnp.zeros_like(mse_scr)

    flatb = flatb_scr[...]
    b1_half = jnp.where(j == 0, b1_ref[pl.ds(e, 1), :_HH],
                        b1_ref[pl.ds(e, 1), _HH:])
    hj = jnp.dot(flatb, w1_ref[0].astype(jnp.bfloat16),
                 preferred_element_type=jnp.float32) + b1_half
    h_scr[:, pl.ds(j * _HH, _HH)] = jnp.maximum(hj, 0.0).astype(jnp.bfloat16)

    @pl.when(j == 1)
    def _finish_expert():
        h = h_scr[...]
        b2 = b2_ref[pl.ds(e, 1), :]
        h2 = jnp.dot(h, w2_ref[0].astype(jnp.bfloat16),
                     preferred_element_type=jnp.float32) + b2
        h2 = jnp.maximum(h2, 0.0)
        b3 = b3_ref[pl.ds(e, 1), :]
        y = jnp.dot(h2.astype(jnp.bfloat16), w3_ref[0].astype(jnp.bfloat16),
                    preferred_element_type=jnp.float32) + b3
        mse_e = jnp.mean((y - tgt_scr[...]) ** 2, axis=-1)
        onehot = (jax.lax.broadcasted_iota(jnp.int32, (1, _E), 1) == e
                  ).astype(jnp.float32)
        mse_scr[...] += mse_e[:, None] * onehot

    @pl.when((e == _E - 1) & (j == 1))
    def _finalize():
        g = g_scr[...]
        B = g.shape[0]
        weighted = jnp.sum(g * mse_scr[...]) / B
        usage = jnp.sum(g, axis=0, keepdims=True) / B
        kl = jnp.sum(usage * (jnp.log(usage + 1e-9) - jnp.log(1.0 / _E)))
        loss_ref[...] = jnp.reshape(weighted + _KL_W * kl, (1, 1))


def kernel(x, t, Wg, bg, W1, b1, W2, b2, W3, b3, Wt):
    B = x.shape[0]
    t_arr = jnp.asarray(t, jnp.int32).reshape(1)
    bg2 = bg.reshape(1, _E)

    loss, idx = pl.pallas_call(
        _moe_kernel,
        grid=(_E, 2),
        in_specs=[
            pl.BlockSpec(memory_space=pltpu.SMEM),
            pl.BlockSpec(memory_space=pltpu.HBM),
            pl.BlockSpec((_FLAT, _E), lambda e, j: (0, 0)),
            pl.BlockSpec((1, _E), lambda e, j: (0, 0)),
            pl.BlockSpec((1, _FLAT, _HH), lambda e, j: (e, 0, j)),
            pl.BlockSpec((_E, _H), lambda e, j: (0, 0)),
            pl.BlockSpec((1, _H, _H), lambda e, j: (e, 0, 0)),
            pl.BlockSpec((_E, _H), lambda e, j: (0, 0)),
            pl.BlockSpec((1, _H, _D), lambda e, j: (e, 0, 0)),
            pl.BlockSpec((_E, _D), lambda e, j: (0, 0)),
            pl.BlockSpec((_NB * _D, _D), lambda e, j: (0, 0)),
        ],
        out_specs=[
            pl.BlockSpec((1, 1), lambda e, j: (0, 0)),
            pl.BlockSpec((1, B), lambda e, j: (0, 0)),
        ],
        out_shape=[
            jax.ShapeDtypeStruct((1, 1), jnp.float32),
            jax.ShapeDtypeStruct((1, B), jnp.int32),
        ],
        scratch_shapes=[
            pltpu.VMEM((B, _FLAT), jnp.float32),
            pltpu.VMEM((B, _FLAT), jnp.bfloat16),
            pltpu.VMEM((B, _H), jnp.bfloat16),
            pltpu.VMEM((B, _E), jnp.float32),
            pltpu.VMEM((B, _E), jnp.float32),
            pltpu.VMEM((B, _D), jnp.float32),
            pltpu.SemaphoreType.DMA,
        ],
        compiler_params=pltpu.CompilerParams(
            dimension_semantics=("arbitrary", "arbitrary"),
        ),
    )(t_arr, x, Wg, bg2, W1, b1, W2, b2, W3, b3, Wt)
    return loss.reshape(()), idx.reshape(B)
